# Initial kernel scaffold; baseline (speedup 1.0000x reference)
import jax
import jax.numpy as jnp
from jax.experimental import pallas as pl


def kernel(q, k, v, w_qs, w_ks, w_vs, ln_a, ln_b):
    raise NotImplementedError("write your pallas kernel here")


# speedup vs baseline: 2.0633x; 2.0633x over previous
import functools
import math

import jax
import jax.numpy as jnp
from jax import lax
from jax.experimental import pallas as pl
from jax.experimental.pallas import tpu as pltpu


def _project_kv_kernel(k_ref, v_ref, wk_ref, wv_ref, mk_ref, mv_ref):
    mk_ref[...] = jnp.dot(
        k_ref[...].astype(jnp.bfloat16), wk_ref[...],
        preferred_element_type=jnp.float32).astype(jnp.bfloat16)
    mv_ref[...] = jnp.dot(
        v_ref[...].astype(jnp.bfloat16), wv_ref[...],
        preferred_element_type=jnp.float32).astype(jnp.bfloat16)


def _attention_kernel(q_ref, mk_ref, mv_ref, wq_ref, ln_ref, o_ref, *, eps):
    qf = q_ref[...]
    mq = jnp.dot(qf.astype(jnp.bfloat16), wq_ref[...],
                 preferred_element_type=jnp.float32).astype(jnp.bfloat16)
    s = lax.dot_general(mq, mk_ref[...], (((1,), (1,)), ((), ())),
                        preferred_element_type=jnp.float32)
    m = jnp.max(s, axis=-1, keepdims=True)
    p = jnp.exp(s - m)
    l = jnp.sum(p, axis=-1, keepdims=True)
    o = jnp.dot(p.astype(jnp.bfloat16), mv_ref[...],
                preferred_element_type=jnp.float32)
    z = o / l + qf
    d = z.shape[-1]
    mu = jnp.mean(z, axis=-1, keepdims=True)
    sigma = jnp.sqrt(
        jnp.sum((z - mu) ** 2, axis=-1, keepdims=True) * (1.0 / (d - 1)))
    z = (z - mu) / (sigma + eps) * ln_ref[0] + ln_ref[1]
    o_ref[...] = z.astype(o_ref.dtype)


def kernel(q, k, v, w_qs, w_ks, w_vs, ln_a, ln_b):
    eps = 1e-3
    L, D = q.shape
    kd = w_qs.shape[1]
    bkv = min(1024, L)
    bq = min(512, L)

    wq = (w_qs * (1.0 / math.sqrt(D))).astype(jnp.bfloat16)
    wk = w_ks.astype(jnp.bfloat16)
    wv = w_vs.astype(jnp.bfloat16)
    ln = jnp.stack(
        [jnp.reshape(ln_a, (D,)), jnp.reshape(ln_b, (D,))], axis=0
    ).astype(jnp.float32)

    mk, mv = pl.pallas_call(
        _project_kv_kernel,
        grid=(L // bkv,),
        in_specs=[
            pl.BlockSpec((bkv, D), lambda i: (i, 0)),
            pl.BlockSpec((bkv, D), lambda i: (i, 0)),
            pl.BlockSpec((D, kd), lambda i: (0, 0)),
            pl.BlockSpec((D, kd), lambda i: (0, 0)),
        ],
        out_specs=[
            pl.BlockSpec((bkv, kd), lambda i: (i, 0)),
            pl.BlockSpec((bkv, kd), lambda i: (i, 0)),
        ],
        out_shape=[
            jax.ShapeDtypeStruct((L, kd), jnp.bfloat16),
            jax.ShapeDtypeStruct((L, kd), jnp.bfloat16),
        ],
        compiler_params=pltpu.CompilerParams(
            dimension_semantics=("parallel",),
        ),
    )(k, v, wk, wv)

    return pl.pallas_call(
        functools.partial(_attention_kernel, eps=eps),
        grid=(L // bq,),
        in_specs=[
            pl.BlockSpec((bq, D), lambda i: (i, 0)),
            pl.BlockSpec((L, kd), lambda i: (0, 0)),
            pl.BlockSpec((L, kd), lambda i: (0, 0)),
            pl.BlockSpec((D, kd), lambda i: (0, 0)),
            pl.BlockSpec((2, D), lambda i: (0, 0)),
        ],
        out_specs=pl.BlockSpec((bq, D), lambda i: (i, 0)),
        out_shape=jax.ShapeDtypeStruct((L, D), jnp.float32),
        compiler_params=pltpu.CompilerParams(
            dimension_semantics=("parallel",),
            vmem_limit_bytes=96 * 1024 * 1024,
        ),
    )(q, mk, mv, wq, ln)


# speedup vs baseline: 2.4313x; 1.1784x over previous
import functools
import math

import jax
import jax.numpy as jnp
from jax import lax
from jax.experimental import pallas as pl
from jax.experimental.pallas import tpu as pltpu


def _fused_kernel(q_ref, k_ref, v_ref, wq_ref, wk_ref, wv_ref,
                  lna_ref, lnb_ref, o_ref, mk_sc, mv_sc, *, eps, scale):
    @pl.when(pl.program_id(0) == 0)
    def _project_kv():
        mk_sc[...] = jnp.dot(
            k_ref[...].astype(jnp.bfloat16), wk_ref[...].astype(jnp.bfloat16),
            preferred_element_type=jnp.float32).astype(jnp.bfloat16)
        mv_sc[...] = jnp.dot(
            v_ref[...].astype(jnp.bfloat16), wv_ref[...].astype(jnp.bfloat16),
            preferred_element_type=jnp.float32).astype(jnp.bfloat16)

    qf = q_ref[...]
    mq = (jnp.dot(qf.astype(jnp.bfloat16), wq_ref[...].astype(jnp.bfloat16),
                  preferred_element_type=jnp.float32)
          * scale).astype(jnp.bfloat16)
    s = lax.dot_general(mq, mk_sc[...], (((1,), (1,)), ((), ())),
                        preferred_element_type=jnp.float32)
    m = jnp.max(s, axis=-1, keepdims=True)
    p = jnp.exp(s - m)
    l = jnp.sum(p, axis=-1, keepdims=True)
    o = jnp.dot(p.astype(jnp.bfloat16), mv_sc[...],
                preferred_element_type=jnp.float32)
    z = o / l + qf
    d = z.shape[-1]
    mu = jnp.mean(z, axis=-1, keepdims=True)
    sigma = jnp.sqrt(
        jnp.sum((z - mu) ** 2, axis=-1, keepdims=True) * (1.0 / (d - 1)))
    z = (z - mu) / (sigma + eps) * lna_ref[...] + lnb_ref[...]
    o_ref[...] = z.astype(o_ref.dtype)


def kernel(q, k, v, w_qs, w_ks, w_vs, ln_a, ln_b):
    eps = 1e-3
    L, D = q.shape
    kd = w_qs.shape[1]
    bq = min(512, L)

    lna = jnp.reshape(ln_a, (1, D)).astype(jnp.float32)
    lnb = jnp.reshape(ln_b, (1, D)).astype(jnp.float32)

    return pl.pallas_call(
        functools.partial(_fused_kernel, eps=eps,
                          scale=1.0 / math.sqrt(D)),
        grid=(L // bq,),
        in_specs=[
            pl.BlockSpec((bq, D), lambda i: (i, 0)),
            pl.BlockSpec((L, D), lambda i: (0, 0)),
            pl.BlockSpec((L, D), lambda i: (0, 0)),
            pl.BlockSpec((D, kd), lambda i: (0, 0)),
            pl.BlockSpec((D, kd), lambda i: (0, 0)),
            pl.BlockSpec((D, kd), lambda i: (0, 0)),
            pl.BlockSpec((1, D), lambda i: (0, 0)),
            pl.BlockSpec((1, D), lambda i: (0, 0)),
        ],
        out_specs=pl.BlockSpec((bq, D), lambda i: (i, 0)),
        out_shape=jax.ShapeDtypeStruct((L, D), jnp.float32),
        scratch_shapes=[
            pltpu.VMEM((L, kd), jnp.bfloat16),
            pltpu.VMEM((L, kd), jnp.bfloat16),
        ],
        compiler_params=pltpu.CompilerParams(
            dimension_semantics=("arbitrary",),
            vmem_limit_bytes=100 * 1024 * 1024,
        ),
    )(q, k, v, w_qs, w_ks, w_vs, lna, lnb)


# speedup vs baseline: 2.4570x; 1.0106x over previous
import functools
import math

import jax
import jax.numpy as jnp
from jax import lax
from jax.experimental import pallas as pl
from jax.experimental.pallas import tpu as pltpu


def _fused_kernel(q_ref, k_ref, v_ref, wq_ref, wk_ref, wv_ref,
                  lna_ref, lnb_ref, o_ref, mk_sc, mv_sc,
                  *, eps, scale, n_p, bkv):
    i = pl.program_id(0)

    @pl.when(i < n_p)
    def _project_kv():
        mk_sc[pl.ds(i * bkv, bkv), :] = jnp.dot(
            k_ref[...].astype(jnp.bfloat16), wk_ref[...].astype(jnp.bfloat16),
            preferred_element_type=jnp.float32).astype(jnp.bfloat16)
        mv_sc[pl.ds(i * bkv, bkv), :] = jnp.dot(
            v_ref[...].astype(jnp.bfloat16), wv_ref[...].astype(jnp.bfloat16),
            preferred_element_type=jnp.float32).astype(jnp.bfloat16)

    @pl.when(i >= n_p)
    def _attend():
        qf = q_ref[...]
        mq = (jnp.dot(qf.astype(jnp.bfloat16),
                      wq_ref[...].astype(jnp.bfloat16),
                      preferred_element_type=jnp.float32)
              * scale).astype(jnp.bfloat16)
        s = lax.dot_general(mq, mk_sc[...], (((1,), (1,)), ((), ())),
                            preferred_element_type=jnp.float32)
        m = jnp.max(s, axis=-1, keepdims=True)
        p = jnp.exp(s - m)
        l = jnp.sum(p, axis=-1, keepdims=True)
        o = jnp.dot(p.astype(jnp.bfloat16), mv_sc[...],
                    preferred_element_type=jnp.float32)
        z = o / l + qf
        d = z.shape[-1]
        mu = jnp.mean(z, axis=-1, keepdims=True)
        sigma = jnp.sqrt(
            jnp.sum((z - mu) ** 2, axis=-1, keepdims=True) * (1.0 / (d - 1)))
        o_ref[...] = ((z - mu) / (sigma + eps) * lna_ref[...]
                      + lnb_ref[...]).astype(o_ref.dtype)


def kernel(q, k, v, w_qs, w_ks, w_vs, ln_a, ln_b):
    eps = 1e-3
    L, D = q.shape
    kd = w_qs.shape[1]
    bq = min(512, L)
    bkv = min(1024, L)
    n_q = L // bq
    n_p = L // bkv

    lna = jnp.reshape(ln_a, (1, D)).astype(jnp.float32)
    lnb = jnp.reshape(ln_b, (1, D)).astype(jnp.float32)

    def kv_idx(i):
        return (jnp.minimum(i, n_p - 1), 0)

    def q_idx(i):
        return (jnp.maximum(i - n_p, 0), 0)

    return pl.pallas_call(
        functools.partial(_fused_kernel, eps=eps, scale=1.0 / math.sqrt(D),
                          n_p=n_p, bkv=bkv),
        grid=(n_p + n_q,),
        in_specs=[
            pl.BlockSpec((bq, D), q_idx),
            pl.BlockSpec((bkv, D), kv_idx),
            pl.BlockSpec((bkv, D), kv_idx),
            pl.BlockSpec((D, kd), lambda i: (0, 0)),
            pl.BlockSpec((D, kd), lambda i: (0, 0)),
            pl.BlockSpec((D, kd), lambda i: (0, 0)),
            pl.BlockSpec((1, D), lambda i: (0, 0)),
            pl.BlockSpec((1, D), lambda i: (0, 0)),
        ],
        out_specs=pl.BlockSpec((bq, D), q_idx),
        out_shape=jax.ShapeDtypeStruct((L, D), jnp.float32),
        scratch_shapes=[
            pltpu.VMEM((L, kd), jnp.bfloat16),
            pltpu.VMEM((L, kd), jnp.bfloat16),
        ],
        compiler_params=pltpu.CompilerParams(
            dimension_semantics=("arbitrary",),
            vmem_limit_bytes=100 * 1024 * 1024,
        ),
    )(q, k, v, w_qs, w_ks, w_vs, lna, lnb)


# speedup vs baseline: 2.5929x; 1.0553x over previous
import functools
import math

import jax
import jax.numpy as jnp
from jax import lax
from jax.experimental import pallas as pl
from jax.experimental.pallas import tpu as pltpu


def _fused_kernel(q_ref, k_ref, v_ref, wq_ref, wk_ref, wv_ref,
                  lna_ref, lnb_ref, o_ref, mk_sc, mv_sc,
                  *, eps, scale, n_p, bkv, n_sub):
    i = pl.program_id(0)

    @pl.when(i < n_p)
    def _project_kv():
        mk_sc[pl.ds(i * bkv, bkv), :] = jnp.dot(
            k_ref[...].astype(jnp.bfloat16), wk_ref[...].astype(jnp.bfloat16),
            preferred_element_type=jnp.float32).astype(jnp.bfloat16)
        mv_sc[pl.ds(i * bkv, bkv), :] = jnp.dot(
            v_ref[...].astype(jnp.bfloat16), wv_ref[...].astype(jnp.bfloat16),
            preferred_element_type=jnp.float32).astype(jnp.bfloat16)

    @pl.when(i >= n_p)
    def _attend():
        bq, d = q_ref.shape
        sub = bq // n_sub
        for h in range(n_sub):
            qf = q_ref[pl.ds(h * sub, sub), :]
            mq = (jnp.dot(qf.astype(jnp.bfloat16),
                          wq_ref[...].astype(jnp.bfloat16),
                          preferred_element_type=jnp.float32)
                  * scale).astype(jnp.bfloat16)
            s = lax.dot_general(mq, mk_sc[...], (((1,), (1,)), ((), ())),
                                preferred_element_type=jnp.float32)
            m = jnp.max(s, axis=-1, keepdims=True)
            p = jnp.exp(s - m)
            l = jnp.sum(p, axis=-1, keepdims=True)
            o = jnp.dot(p.astype(jnp.bfloat16), mv_sc[...],
                        preferred_element_type=jnp.float32)
            z = o / l + qf
            mu = jnp.mean(z, axis=-1, keepdims=True)
            sigma = jnp.sqrt(
                jnp.sum((z - mu) ** 2, axis=-1, keepdims=True)
                * (1.0 / (d - 1)))
            o_ref[pl.ds(h * sub, sub), :] = (
                (z - mu) / (sigma + eps) * lna_ref[...]
                + lnb_ref[...]).astype(o_ref.dtype)


def kernel(q, k, v, w_qs, w_ks, w_vs, ln_a, ln_b):
    eps = 1e-3
    L, D = q.shape
    kd = w_qs.shape[1]
    bq = min(1024, L)
    n_sub = 2 if bq >= 1024 else 1
    bkv = min(1024, L)
    n_q = L // bq
    n_p = L // bkv

    lna = jnp.reshape(ln_a, (1, D)).astype(jnp.float32)
    lnb = jnp.reshape(ln_b, (1, D)).astype(jnp.float32)

    def kv_idx(i):
        return (jnp.minimum(i, n_p - 1), 0)

    def q_idx(i):
        return (jnp.maximum(i - n_p, 0), 0)

    return pl.pallas_call(
        functools.partial(_fused_kernel, eps=eps, scale=1.0 / math.sqrt(D),
                          n_p=n_p, bkv=bkv, n_sub=n_sub),
        grid=(n_p + n_q,),
        in_specs=[
            pl.BlockSpec((bq, D), q_idx),
            pl.BlockSpec((bkv, D), kv_idx),
            pl.BlockSpec((bkv, D), kv_idx),
            pl.BlockSpec((D, kd), lambda i: (0, 0)),
            pl.BlockSpec((D, kd), lambda i: (0, 0)),
            pl.BlockSpec((D, kd), lambda i: (0, 0)),
            pl.BlockSpec((1, D), lambda i: (0, 0)),
            pl.BlockSpec((1, D), lambda i: (0, 0)),
        ],
        out_specs=pl.BlockSpec((bq, D), q_idx),
        out_shape=jax.ShapeDtypeStruct((L, D), jnp.float32),
        scratch_shapes=[
            pltpu.VMEM((L, kd), jnp.bfloat16),
            pltpu.VMEM((L, kd), jnp.bfloat16),
        ],
        compiler_params=pltpu.CompilerParams(
            dimension_semantics=("arbitrary",),
            vmem_limit_bytes=100 * 1024 * 1024,
        ),
    )(q, k, v, w_qs, w_ks, w_vs, lna, lnb)


# speedup vs baseline: 2.6467x; 1.0207x over previous
import functools
import math

import jax
import jax.numpy as jnp
from jax import lax
from jax.experimental import pallas as pl
from jax.experimental.pallas import tpu as pltpu


def _fused_kernel(q_ref, k_ref, v_ref, wq_ref, wk_ref, wv_ref,
                  lna_ref, lnb_ref, o_ref, mk_sc, mv_sc,
                  *, eps, scale, n_p, bkv, n_sub):
    i = pl.program_id(0)

    @pl.when(i < n_p)
    def _project_kv():
        mk_sc[pl.ds(i * bkv, bkv), :] = jnp.dot(
            k_ref[...].astype(jnp.bfloat16), wk_ref[...].astype(jnp.bfloat16),
            preferred_element_type=jnp.float32).astype(jnp.bfloat16)
        mv_sc[pl.ds(i * bkv, bkv), :] = jnp.dot(
            v_ref[...].astype(jnp.bfloat16), wv_ref[...].astype(jnp.bfloat16),
            preferred_element_type=jnp.float32).astype(jnp.bfloat16)

    @pl.when(i >= n_p)
    def _attend():
        bq, d = q_ref.shape
        sub = bq // n_sub
        for h in range(n_sub):
            qf = q_ref[pl.ds(h * sub, sub), :]
            mq = (jnp.dot(qf.astype(jnp.bfloat16),
                          wq_ref[...].astype(jnp.bfloat16),
                          preferred_element_type=jnp.float32)
                  * scale).astype(jnp.bfloat16)
            s = lax.dot_general(mq, mk_sc[...], (((1,), (1,)), ((), ())),
                                preferred_element_type=jnp.float32)
            m = jnp.max(s, axis=-1, keepdims=True)
            p = jnp.exp(s - m)
            l = jnp.sum(p, axis=-1, keepdims=True)
            o = jnp.dot(p.astype(jnp.bfloat16), mv_sc[...],
                        preferred_element_type=jnp.float32)
            z = o / l + qf
            mu = jnp.mean(z, axis=-1, keepdims=True)
            sigma = jnp.sqrt(
                jnp.sum((z - mu) ** 2, axis=-1, keepdims=True)
                * (1.0 / (d - 1)))
            o_ref[pl.ds(h * sub, sub), :] = (
                (z - mu) / (sigma + eps) * lna_ref[...]
                + lnb_ref[...]).astype(o_ref.dtype)


def kernel(q, k, v, w_qs, w_ks, w_vs, ln_a, ln_b):
    eps = 1e-3
    L, D = q.shape
    kd = w_qs.shape[1]
    bq = min(2048, L)
    n_sub = bq // 512 if bq >= 1024 else 1
    bkv = min(1024, L)
    n_q = L // bq
    n_p = L // bkv

    lna = jnp.reshape(ln_a, (1, D)).astype(jnp.float32)
    lnb = jnp.reshape(ln_b, (1, D)).astype(jnp.float32)

    def kv_idx(i):
        return (jnp.minimum(i, n_p - 1), 0)

    def q_idx(i):
        return (jnp.maximum(i - n_p, 0), 0)

    return pl.pallas_call(
        functools.partial(_fused_kernel, eps=eps, scale=1.0 / math.sqrt(D),
                          n_p=n_p, bkv=bkv, n_sub=n_sub),
        grid=(n_p + n_q,),
        in_specs=[
            pl.BlockSpec((bq, D), q_idx),
            pl.BlockSpec((bkv, D), kv_idx),
            pl.BlockSpec((bkv, D), kv_idx),
            pl.BlockSpec((D, kd), lambda i: (0, 0)),
            pl.BlockSpec((D, kd), lambda i: (0, 0)),
            pl.BlockSpec((D, kd), lambda i: (0, 0)),
            pl.BlockSpec((1, D), lambda i: (0, 0)),
            pl.BlockSpec((1, D), lambda i: (0, 0)),
        ],
        out_specs=pl.BlockSpec((bq, D), q_idx),
        out_shape=jax.ShapeDtypeStruct((L, D), jnp.float32),
        scratch_shapes=[
            pltpu.VMEM((L, kd), jnp.bfloat16),
            pltpu.VMEM((L, kd), jnp.bfloat16),
        ],
        compiler_params=pltpu.CompilerParams(
            dimension_semantics=("arbitrary",),
            vmem_limit_bytes=100 * 1024 * 1024,
        ),
    )(q, k, v, w_qs, w_ks, w_vs, lna, lnb)
